# Initial kernel scaffold; baseline (speedup 1.0000x reference)
import jax
import jax.numpy as jnp
from jax import lax
from jax.experimental import pallas as pl
from jax.experimental.pallas import tpu as pltpu
from jax.experimental.pallas import tpu_sc as plsc

N = 10000
E = 320000
D = 128
H = 8
K = 16
FFN = 512

NC = 2
NS = 16
NW = NC * NS
GRP = 128
N_PAD = 10240
EPW = 10240
E_PAD = NW * EPW
STRIPE = N_PAD // NS
NEG = -3.0e38


def _proj_body(x_ref, wf_ref, a_ref, wh_ref, ea_ref):
    x = x_ref[...]
    wh = jnp.dot(x, wf_ref[...], preferred_element_type=jnp.float32)
    wh_ref[...] = wh
    ea_ref[...] = jnp.dot(wh, a_ref[...], preferred_element_type=jnp.float32)


def _proj(x, w_flat, a_mat):
    R = 400
    return pl.pallas_call(
        _proj_body,
        grid=(N // R,),
        in_specs=[
            pl.BlockSpec((R, D), lambda i: (i, 0)),
            pl.BlockSpec((D, D), lambda i: (0, 0)),
            pl.BlockSpec((D, 2 * H), lambda i: (0, 0)),
        ],
        out_specs=[
            pl.BlockSpec((R, D), lambda i: (i, 0)),
            pl.BlockSpec((R, 2 * H), lambda i: (i, 0)),
        ],
        out_shape=[
            jax.ShapeDtypeStruct((N, D), jnp.float32),
            jax.ShapeDtypeStruct((N, 2 * H), jnp.float32),
        ],
    )(x, w_flat, a_mat)


def _segmax_body(src_hbm, dst_hbm, ea_hbm, e_out, m0_out, m1_out,
                 macc, sidx, didx, rows_s, rows_d, ebuf, mtmp, tbuf):
    c = lax.axis_index("c")
    s_ = lax.axis_index("s")
    wid = c * NS + s_
    lo = wid * EPW
    ng = jnp.maximum(0, jnp.minimum(EPW, E - lo)) // GRP

    lane = lax.iota(jnp.int32, 16)
    hsel = lane & 7
    sh8 = (lane + 8) & 15
    lane8 = lane < 8
    neg16 = jnp.full((16,), NEG, jnp.float32)

    def _init(j, _):
        for r in range(H):
            macc[r, pl.ds(j * 16, 16)] = neg16
        return 0
    lax.fori_loop(0, N_PAD // 16, _init, 0)

    def _group(g, _):
        base = lo + g * GRP
        pltpu.sync_copy(src_hbm.at[pl.ds(base, GRP)], sidx)
        pltpu.sync_copy(dst_hbm.at[pl.ds(base, GRP)], didx)
        pltpu.sync_copy(ea_hbm.at[sidx], rows_s)
        pltpu.sync_copy(ea_hbm.at[didx], rows_d)

        def _sub(jj, _):
            dvec = didx[pl.ds(jj * 16, 16)]
            for jl in range(16):
                row = jj * 16 + jl
                rs = rows_s[row, :]
                rd = rows_d[row, :]
                ef = rs + jnp.take(rd, sh8, mode="promise_in_bounds")
                lk = jnp.where(ef >= 0.0, ef, 0.2 * ef)
                ebuf[row, :] = lk
                dsp = jnp.take(dvec, jnp.full((16,), jl, jnp.int32),
                               mode="promise_in_bounds")
                cur = plsc.load_gather(macc, [hsel, dsp])
                plsc.store_scatter(macc, [hsel, dsp],
                                   jnp.maximum(cur, lk), mask=lane8)
            return 0
        lax.fori_loop(0, GRP // 16, _sub, 0)
        pltpu.sync_copy(ebuf, e_out.at[pl.ds(base, GRP)])
        return 0
    lax.fori_loop(0, ng, _group, 0)

    pltpu.sync_copy(macc, mtmp.at[s_])
    plsc.subcore_barrier()

    sbase = s_ * STRIPE
    pltpu.sync_copy(mtmp.at[0, :, pl.ds(sbase, STRIPE)],
                    macc.at[:, pl.ds(0, STRIPE)])
    for p in range(1, NS):
        pltpu.sync_copy(mtmp.at[p, :, pl.ds(sbase, STRIPE)],
                        macc.at[:, pl.ds(STRIPE, STRIPE)])

        def _mrg(j, _):
            for r in range(H):
                a = macc[r, pl.ds(j * 16, 16)]
                b = macc[r, pl.ds(STRIPE + j * 16, 16)]
                macc[r, pl.ds(j * 16, 16)] = jnp.maximum(a, b)
            return 0
        lax.fori_loop(0, STRIPE // 16, _mrg, 0)

    def _tr(nb, _):
        for jl in range(16):
            node = nb * 16 + jl
            vec = plsc.load_gather(
                macc, [hsel, jnp.full((16,), node, jnp.int32)])
            tbuf[jl, :] = vec

        @pl.when(c == 0)
        def _():
            pltpu.sync_copy(tbuf, m0_out.at[pl.ds(sbase + nb * 16, 16)])

        @pl.when(c == 1)
        def _():
            pltpu.sync_copy(tbuf, m1_out.at[pl.ds(sbase + nb * 16, 16)])
        return 0
    lax.fori_loop(0, STRIPE // 16, _tr, 0)


def _sc_segmax(src_p, dst_p, e_all):
    mesh = plsc.VectorSubcoreMesh(core_axis_name="c", subcore_axis_name="s")
    f = pl.kernel(
        _segmax_body,
        out_type=[
            jax.ShapeDtypeStruct((E_PAD, 16), jnp.float32),
            jax.ShapeDtypeStruct((N_PAD, 16), jnp.float32),
            jax.ShapeDtypeStruct((N_PAD, 16), jnp.float32),
        ],
        mesh=mesh,
        scratch_types=[
            pltpu.VMEM((H, N_PAD), jnp.float32),
            pltpu.VMEM((GRP,), jnp.int32),
            pltpu.VMEM((GRP,), jnp.int32),
            pltpu.VMEM((GRP, 16), jnp.float32),
            pltpu.VMEM((GRP, 16), jnp.float32),
            pltpu.VMEM((GRP, 16), jnp.float32),
            pltpu.VMEM_SHARED((NS, H, N_PAD), jnp.float32),
            pltpu.VMEM((16, 16), jnp.float32),
        ],
    )
    return f(src_p, dst_p, e_all)


def _msg_body(src_hbm, dst_hbm, e_hbm, m0_hbm, m1_hbm, wh_hbm,
              num_out, den_out,
              sidx, didx, whbuf, ebuf, m0buf, m1buf, exbuf, nstage,
              num_sh, den_sh):
    c = lax.axis_index("c")
    s_ = lax.axis_index("s")
    wid = c * NS + s_
    lo = wid * EPW
    ng = jnp.maximum(0, jnp.minimum(EPW, E - lo)) // GRP

    z16 = jnp.zeros((16,), jnp.float32)

    def _z(r, _):
        for cb in range(H):
            nstage[r, pl.ds(cb * 16, 16)] = z16
        exbuf[r, :] = z16
        return 0
    lax.fori_loop(0, GRP, _z, 0)
    sbase = s_ * STRIPE
    for t in range(STRIPE // GRP):
        pltpu.sync_copy(nstage, num_sh.at[pl.ds(sbase + t * GRP, GRP)])
        pltpu.sync_copy(exbuf, den_sh.at[pl.ds(sbase + t * GRP, GRP)])
    plsc.subcore_barrier()

    def _group(g, _):
        base = lo + g * GRP
        pltpu.sync_copy(src_hbm.at[pl.ds(base, GRP)], sidx)
        pltpu.sync_copy(dst_hbm.at[pl.ds(base, GRP)], didx)
        pltpu.sync_copy(wh_hbm.at[sidx], whbuf)
        pltpu.sync_copy(e_hbm.at[pl.ds(base, GRP)], ebuf)
        pltpu.sync_copy(m0_hbm.at[didx], m0buf)
        pltpu.sync_copy(m1_hbm.at[didx], m1buf)

        def _sub(jj, _):
            for jl in range(16):
                row = jj * 16 + jl
                ev = ebuf[row, :]
                mv = jnp.maximum(m0buf[row, :], m1buf[row, :])
                ex = jnp.exp(jnp.minimum(ev - mv, 0.0))
                exbuf[row, :] = ex
                for h in range(H):
                    exh = jnp.take(ex, jnp.full((16,), h, jnp.int32),
                                   mode="promise_in_bounds")
                    nstage[row, pl.ds(h * 16, 16)] = (
                        whbuf[row, pl.ds(h * 16, 16)] * exh)
            return 0
        lax.fori_loop(0, GRP // 16, _sub, 0)

        pltpu.sync_copy(nstage, num_sh.at[didx], add=True)
        pltpu.sync_copy(exbuf, den_sh.at[didx], add=True)
        return 0
    lax.fori_loop(0, ng, _group, 0)

    plsc.subcore_barrier()
    @pl.when(c == 0)
    def _():
        pltpu.sync_copy(num_sh.at[pl.ds(sbase, STRIPE)],
                        num_out.at[0, pl.ds(sbase, STRIPE)])
        pltpu.sync_copy(den_sh.at[pl.ds(sbase, STRIPE)],
                        den_out.at[0, pl.ds(sbase, STRIPE)])

    @pl.when(c == 1)
    def _():
        pltpu.sync_copy(num_sh.at[pl.ds(sbase, STRIPE)],
                        num_out.at[1, pl.ds(sbase, STRIPE)])
        pltpu.sync_copy(den_sh.at[pl.ds(sbase, STRIPE)],
                        den_out.at[1, pl.ds(sbase, STRIPE)])


def _sc_msg(src_p, dst_p, e_edge, m0, m1, wh):
    mesh = plsc.VectorSubcoreMesh(core_axis_name="c", subcore_axis_name="s")
    f = pl.kernel(
        _msg_body,
        out_type=[
            jax.ShapeDtypeStruct((NC, N_PAD, D), jnp.float32),
            jax.ShapeDtypeStruct((NC, N_PAD, 16), jnp.float32),
        ],
        mesh=mesh,
        scratch_types=[
            pltpu.VMEM((GRP,), jnp.int32),
            pltpu.VMEM((GRP,), jnp.int32),
            pltpu.VMEM((GRP, D), jnp.float32),
            pltpu.VMEM((GRP, 16), jnp.float32),
            pltpu.VMEM((GRP, 16), jnp.float32),
            pltpu.VMEM((GRP, 16), jnp.float32),
            pltpu.VMEM((GRP, 16), jnp.float32),
            pltpu.VMEM((GRP, D), jnp.float32),
            pltpu.VMEM_SHARED((N_PAD, D), jnp.float32),
            pltpu.VMEM_SHARED((N_PAD, 16), jnp.float32),
        ],
    )
    return f(src_p, dst_p, e_edge, m0, m1, wh)


def _ffn_body(num_ref, den_ref, s_ref, lng_ref, lnb_ref,
              w1_ref, b1_ref, w2_ref, b2_ref, out_ref):
    num = num_ref[0] + num_ref[1]
    den = den_ref[0] + den_ref[1]
    den8 = den[:, 0:H]
    rr = lax.broadcasted_iota(jnp.int32, (H, D), 0)
    cc = lax.broadcasted_iota(jnp.int32, (H, D), 1)
    bmat = (rr == cc // K).astype(jnp.float32)
    denb = jnp.dot(den8, bmat, preferred_element_type=jnp.float32)
    gat = num / (denb + 1e-10)
    gat = jnp.where(gat > 0.0, gat, jnp.exp(jnp.minimum(gat, 0.0)) - 1.0)
    h = gat + s_ref[...]
    mu = jnp.mean(h, axis=-1, keepdims=True)
    var = jnp.mean((h - mu) * (h - mu), axis=-1, keepdims=True)
    hn = (h - mu) / jnp.sqrt(var + 1e-6) * lng_ref[...] + lnb_ref[...]
    inter = jnp.maximum(
        jnp.dot(hn, w1_ref[...], preferred_element_type=jnp.float32)
        + b1_ref[...], 0.0)
    out_ref[...] = (jnp.dot(inter, w2_ref[...],
                            preferred_element_type=jnp.float32)
                    + b2_ref[...] + h)


def _ffn_tc(num, den, s, ln_g, ln_b, w1, b1, w2, b2):
    R = 400
    return pl.pallas_call(
        _ffn_body,
        grid=(N // R,),
        in_specs=[
            pl.BlockSpec((NC, R, D), lambda i: (0, i, 0)),
            pl.BlockSpec((NC, R, 16), lambda i: (0, i, 0)),
            pl.BlockSpec((R, D), lambda i: (i, 0)),
            pl.BlockSpec((1, D), lambda i: (0, 0)),
            pl.BlockSpec((1, D), lambda i: (0, 0)),
            pl.BlockSpec((D, FFN), lambda i: (0, 0)),
            pl.BlockSpec((1, FFN), lambda i: (0, 0)),
            pl.BlockSpec((FFN, D), lambda i: (0, 0)),
            pl.BlockSpec((1, D), lambda i: (0, 0)),
        ],
        out_specs=pl.BlockSpec((R, D), lambda i: (i, 0)),
        out_shape=jax.ShapeDtypeStruct((N, D), jnp.float32),
    )(num, den, s, ln_g.reshape(1, D), ln_b.reshape(1, D),
      w1, b1.reshape(1, FFN), w2, b2.reshape(1, D))


def kernel(w, s, edge_index, W, a_src, a_dst, ln_g, ln_b, W1, b1, W2, b2):
    w_flat = W.transpose(1, 0, 2).reshape(D, D)
    eye8 = jnp.eye(H, dtype=jnp.float32)
    a1 = (a_src[:, :, None] * eye8[:, None, :]).reshape(D, H)
    a2 = (a_dst[:, :, None] * eye8[:, None, :]).reshape(D, H)
    a_mat = jnp.concatenate([a1, a2], axis=1)

    src = edge_index[0].astype(jnp.int32)
    dst = edge_index[1].astype(jnp.int32)
    pad = jnp.zeros((E_PAD - E,), jnp.int32)
    src_p = jnp.concatenate([src, pad])
    dst_p = jnp.concatenate([dst, pad])

    wh, e_all = _proj(w, w_flat, a_mat)
    e_edge, m0, m1 = _sc_segmax(src_p, dst_p, e_all)
    num, den = _sc_msg(src_p, dst_p, e_edge, m0, m1, wh)
    return _ffn_tc(num, den, s, ln_g, ln_b, W1, b1, W2, b2)


# speedup vs baseline: 28.4024x; 28.4024x over previous
import jax
import jax.numpy as jnp
from jax import lax
from jax.experimental import pallas as pl
from jax.experimental.pallas import tpu as pltpu
from jax.experimental.pallas import tpu_sc as plsc

N = 10000
E = 320000
D = 128
H = 8
K = 16
FFN = 512

NC = 2
NS = 16
NW = NC * NS
GRP = 128
GRP2 = 64
N_PAD = 10240
EPW = 10240
E_PAD = NW * EPW
STRIPE = N_PAD // NS
NEG = -3.0e38

_GDN = lax.GatherDimensionNumbers(
    offset_dims=(), collapsed_slice_dims=(0,), start_index_map=(0,))


def _dg(x, idx):
    return lax.gather(x, idx[:, None], _GDN, slice_sizes=(1,),
                      mode=lax.GatherScatterMode.PROMISE_IN_BOUNDS)


def _proj_body(x_ref, wf_ref, a_ref, wh_ref, ea_ref):
    x = x_ref[...]
    wh = jnp.dot(x, wf_ref[...], preferred_element_type=jnp.float32)
    wh_ref[...] = wh
    ea_ref[...] = jnp.dot(wh, a_ref[...], preferred_element_type=jnp.float32)


def _proj(x, w_flat, a_mat):
    R = 400
    return pl.pallas_call(
        _proj_body,
        grid=(N // R,),
        in_specs=[
            pl.BlockSpec((R, D), lambda i: (i, 0)),
            pl.BlockSpec((D, D), lambda i: (0, 0)),
            pl.BlockSpec((D, 2 * H), lambda i: (0, 0)),
        ],
        out_specs=[
            pl.BlockSpec((R, D), lambda i: (i, 0)),
            pl.BlockSpec((R, 2 * H), lambda i: (i, 0)),
        ],
        out_shape=[
            jax.ShapeDtypeStruct((N, D), jnp.float32),
            jax.ShapeDtypeStruct((N, 2 * H), jnp.float32),
        ],
    )(x, w_flat, a_mat)


def _segmax_body(src_hbm, dst_hbm, ea_hbm, e_out, mparts_out, m0_out, m1_out,
                 macc, sidx, didx, rows_s, rows_d, ebuf, mbufa, mbufb, tbuf):
    c = lax.axis_index("c")
    s_ = lax.axis_index("s")
    wid = c * NS + s_
    lo = wid * EPW
    ng = jnp.maximum(0, jnp.minimum(EPW, E - lo)) // GRP

    lane = lax.iota(jnp.int32, 16)
    hsel = lane & 7
    sh8 = (lane + 8) & 15
    lane8 = lane < 8
    neg16 = jnp.full((16,), NEG, jnp.float32)

    def _init(j, _):
        for r in range(H):
            macc[r, pl.ds(j * 16, 16)] = neg16
        return 0
    lax.fori_loop(0, N_PAD // 16, _init, 0)

    def _group(g, _):
        base = lo + g * GRP
        pltpu.sync_copy(src_hbm.at[pl.ds(base, GRP)], sidx)
        pltpu.sync_copy(dst_hbm.at[pl.ds(base, GRP)], didx)
        pltpu.sync_copy(ea_hbm.at[sidx], rows_s)
        pltpu.sync_copy(ea_hbm.at[didx], rows_d)

        def _sub(jj, _):
            dvec = didx[pl.ds(jj * 16, 16)]
            for jl in range(16):
                row = jj * 16 + jl
                rs = rows_s[row, :]
                rd = rows_d[row, :]
                ef = rs + _dg(rd, sh8)
                lk = jnp.where(ef >= 0.0, ef, 0.2 * ef)
                ebuf[row, :] = lk
                dsp = _dg(dvec, jnp.full((16,), jl, jnp.int32))
                cur = plsc.load_gather(macc, [hsel, dsp])
                plsc.store_scatter(macc, [hsel, dsp],
                                   jnp.maximum(cur, lk), mask=lane8)
            return 0
        lax.fori_loop(0, GRP // 16, _sub, 0)
        pltpu.sync_copy(ebuf, e_out.at[pl.ds(base, GRP)])
        return 0
    lax.fori_loop(0, ng, _group, 0)

    pltpu.sync_copy(macc, mparts_out.at[wid])
    plsc.subcore_barrier()

    sbase = s_ * STRIPE
    pltpu.sync_copy(mparts_out.at[c * NS, :, pl.ds(sbase, STRIPE)], mbufa)
    for p in range(1, NS):
        pltpu.sync_copy(mparts_out.at[c * NS + p, :, pl.ds(sbase, STRIPE)],
                        mbufb)

        def _mrg(j, _):
            for r in range(H):
                a = mbufa[r, pl.ds(j * 16, 16)]
                b = mbufb[r, pl.ds(j * 16, 16)]
                mbufa[r, pl.ds(j * 16, 16)] = jnp.maximum(a, b)
            return 0
        lax.fori_loop(0, STRIPE // 16, _mrg, 0)

    def _tr(nb, _):
        for jl in range(16):
            node = nb * 16 + jl
            vec = plsc.load_gather(
                mbufa, [hsel, jnp.full((16,), node, jnp.int32)])
            tbuf[jl, :] = vec

        @pl.when(c == 0)
        def _():
            pltpu.sync_copy(tbuf, m0_out.at[pl.ds(sbase + nb * 16, 16)])

        @pl.when(c == 1)
        def _():
            pltpu.sync_copy(tbuf, m1_out.at[pl.ds(sbase + nb * 16, 16)])
        return 0
    lax.fori_loop(0, STRIPE // 16, _tr, 0)


def _sc_segmax(src_p, dst_p, e_all):
    mesh = plsc.VectorSubcoreMesh(core_axis_name="c", subcore_axis_name="s")
    f = pl.kernel(
        _segmax_body,
        out_type=[
            jax.ShapeDtypeStruct((E_PAD, 16), jnp.float32),
            jax.ShapeDtypeStruct((NW, H, N_PAD), jnp.float32),
            jax.ShapeDtypeStruct((N_PAD, 16), jnp.float32),
            jax.ShapeDtypeStruct((N_PAD, 16), jnp.float32),
        ],
        mesh=mesh,
        compiler_params=pltpu.CompilerParams(use_tc_tiling_on_sc=False, needs_layout_passes=False),
        scratch_types=[
            pltpu.VMEM((H, N_PAD), jnp.float32),
            pltpu.VMEM((GRP,), jnp.int32),
            pltpu.VMEM((GRP,), jnp.int32),
            pltpu.VMEM((GRP, 16), jnp.float32),
            pltpu.VMEM((GRP, 16), jnp.float32),
            pltpu.VMEM((GRP, 16), jnp.float32),
            pltpu.VMEM((H, STRIPE), jnp.float32),
            pltpu.VMEM((H, STRIPE), jnp.float32),
            pltpu.VMEM((16, 16), jnp.float32),
        ],
    )
    e_edge, _mparts, m0, m1 = f(src_p, dst_p, e_all)
    return e_edge, m0, m1


def _msg_body(src_hbm, dst_hbm, e_hbm, m0_hbm, m1_hbm, wh_hbm,
              num_out, den_out,
              sidx, didx, whbuf, ebuf, m0buf, m1buf, exbuf, nstage,
              num_sh, den_sh):
    c = lax.axis_index("c")
    s_ = lax.axis_index("s")
    wid = c * NS + s_
    lo = wid * EPW
    ng = jnp.maximum(0, jnp.minimum(EPW, E - lo)) // GRP2

    z16 = jnp.zeros((16,), jnp.float32)

    def _z(r, _):
        for cb in range(H):
            nstage[r, pl.ds(cb * 16, 16)] = z16
        exbuf[r, :] = z16
        return 0
    lax.fori_loop(0, GRP2, _z, 0)
    sbase = s_ * STRIPE
    for t in range(STRIPE // GRP2):
        pltpu.sync_copy(nstage, num_sh.at[pl.ds(sbase + t * GRP2, GRP2)])
        pltpu.sync_copy(exbuf, den_sh.at[pl.ds(sbase + t * GRP2, GRP2)])
    plsc.subcore_barrier()

    def _group(g, _):
        base = lo + g * GRP2
        pltpu.sync_copy(src_hbm.at[pl.ds(base, GRP2)], sidx)
        pltpu.sync_copy(dst_hbm.at[pl.ds(base, GRP2)], didx)
        pltpu.sync_copy(wh_hbm.at[sidx], whbuf)
        pltpu.sync_copy(e_hbm.at[pl.ds(base, GRP2)], ebuf)
        pltpu.sync_copy(m0_hbm.at[didx], m0buf)
        pltpu.sync_copy(m1_hbm.at[didx], m1buf)

        def _sub(jj, _):
            for jl in range(16):
                row = jj * 16 + jl
                ev = ebuf[row, :]
                mv = jnp.maximum(m0buf[row, :], m1buf[row, :])
                ex = jnp.exp(jnp.minimum(ev - mv, 0.0))
                exbuf[row, :] = ex
                for h in range(H):
                    exh = _dg(ex, jnp.full((16,), h, jnp.int32))
                    nstage[row, pl.ds(h * 16, 16)] = (
                        whbuf[row, pl.ds(h * 16, 16)] * exh)
            return 0
        lax.fori_loop(0, GRP2 // 16, _sub, 0)

        pltpu.sync_copy(nstage, num_sh.at[didx], add=True)
        pltpu.sync_copy(exbuf, den_sh.at[didx], add=True)
        return 0
    lax.fori_loop(0, ng, _group, 0)

    plsc.subcore_barrier()
    @pl.when(c == 0)
    def _():
        pltpu.sync_copy(num_sh.at[pl.ds(sbase, STRIPE)],
                        num_out.at[0, pl.ds(sbase, STRIPE)])
        pltpu.sync_copy(den_sh.at[pl.ds(sbase, STRIPE)],
                        den_out.at[0, pl.ds(sbase, STRIPE)])

    @pl.when(c == 1)
    def _():
        pltpu.sync_copy(num_sh.at[pl.ds(sbase, STRIPE)],
                        num_out.at[1, pl.ds(sbase, STRIPE)])
        pltpu.sync_copy(den_sh.at[pl.ds(sbase, STRIPE)],
                        den_out.at[1, pl.ds(sbase, STRIPE)])


def _sc_msg(src_p, dst_p, e_edge, m0, m1, wh):
    mesh = plsc.VectorSubcoreMesh(core_axis_name="c", subcore_axis_name="s")
    f = pl.kernel(
        _msg_body,
        out_type=[
            jax.ShapeDtypeStruct((NC, N_PAD, D), jnp.float32),
            jax.ShapeDtypeStruct((NC, N_PAD, 16), jnp.float32),
        ],
        mesh=mesh,
        compiler_params=pltpu.CompilerParams(use_tc_tiling_on_sc=False, needs_layout_passes=False),
        scratch_types=[
            pltpu.VMEM((GRP2,), jnp.int32),
            pltpu.VMEM((GRP2,), jnp.int32),
            pltpu.VMEM((GRP2, D), jnp.float32),
            pltpu.VMEM((GRP2, 16), jnp.float32),
            pltpu.VMEM((GRP2, 16), jnp.float32),
            pltpu.VMEM((GRP2, 16), jnp.float32),
            pltpu.VMEM((GRP2, 16), jnp.float32),
            pltpu.VMEM((GRP2, D), jnp.float32),
            pltpu.VMEM_SHARED((N_PAD, D), jnp.float32),
            pltpu.VMEM_SHARED((N_PAD, 16), jnp.float32),
        ],
    )
    return f(src_p, dst_p, e_edge, m0, m1, wh)


def _ffn_body(num_ref, den_ref, s_ref, lng_ref, lnb_ref,
              w1_ref, b1_ref, w2_ref, b2_ref, out_ref):
    num = num_ref[0] + num_ref[1]
    den = den_ref[0] + den_ref[1]
    den8 = den[:, 0:H]
    rr = lax.broadcasted_iota(jnp.int32, (H, D), 0)
    cc = lax.broadcasted_iota(jnp.int32, (H, D), 1)
    bmat = (rr == cc // K).astype(jnp.float32)
    denb = jnp.dot(den8, bmat, preferred_element_type=jnp.float32)
    gat = num / (denb + 1e-10)
    gat = jnp.where(gat > 0.0, gat, jnp.exp(jnp.minimum(gat, 0.0)) - 1.0)
    h = gat + s_ref[...]
    mu = jnp.mean(h, axis=-1, keepdims=True)
    var = jnp.mean((h - mu) * (h - mu), axis=-1, keepdims=True)
    hn = (h - mu) / jnp.sqrt(var + 1e-6) * lng_ref[...] + lnb_ref[...]
    inter = jnp.maximum(
        jnp.dot(hn, w1_ref[...], preferred_element_type=jnp.float32)
        + b1_ref[...], 0.0)
    out_ref[...] = (jnp.dot(inter, w2_ref[...],
                            preferred_element_type=jnp.float32)
                    + b2_ref[...] + h)


def _ffn_tc(num, den, s, ln_g, ln_b, w1, b1, w2, b2):
    R = 400
    return pl.pallas_call(
        _ffn_body,
        grid=(N // R,),
        in_specs=[
            pl.BlockSpec((NC, R, D), lambda i: (0, i, 0)),
            pl.BlockSpec((NC, R, 16), lambda i: (0, i, 0)),
            pl.BlockSpec((R, D), lambda i: (i, 0)),
            pl.BlockSpec((1, D), lambda i: (0, 0)),
            pl.BlockSpec((1, D), lambda i: (0, 0)),
            pl.BlockSpec((D, FFN), lambda i: (0, 0)),
            pl.BlockSpec((1, FFN), lambda i: (0, 0)),
            pl.BlockSpec((FFN, D), lambda i: (0, 0)),
            pl.BlockSpec((1, D), lambda i: (0, 0)),
        ],
        out_specs=pl.BlockSpec((R, D), lambda i: (i, 0)),
        out_shape=jax.ShapeDtypeStruct((N, D), jnp.float32),
    )(num, den, s, ln_g.reshape(1, D), ln_b.reshape(1, D),
      w1, b1.reshape(1, FFN), w2, b2.reshape(1, D))


def kernel(w, s, edge_index, W, a_src, a_dst, ln_g, ln_b, W1, b1, W2, b2):
    w_flat = W.transpose(1, 0, 2).reshape(D, D)
    eye8 = jnp.eye(H, dtype=jnp.float32)
    a1 = (a_src[:, :, None] * eye8[:, None, :]).reshape(D, H)
    a2 = (a_dst[:, :, None] * eye8[:, None, :]).reshape(D, H)
    a_mat = jnp.concatenate([a1, a2], axis=1)

    src = edge_index[0].astype(jnp.int32)
    dst = edge_index[1].astype(jnp.int32)
    pad = jnp.zeros((E_PAD - E,), jnp.int32)
    src_p = jnp.concatenate([src, pad])
    dst_p = jnp.concatenate([dst, pad])

    wh, e_all = _proj(w, w_flat, a_mat)
    e_edge, m0, m1 = _sc_segmax(src_p, dst_p, e_all)
    num, den = _sc_msg(src_p, dst_p, e_edge, m0, m1, wh)
    return _ffn_tc(num, den, s, ln_g, ln_b, W1, b1, W2, b2)


# speedup vs baseline: 38.9015x; 1.3697x over previous
import jax
import jax.numpy as jnp
from jax import lax
from jax.experimental import pallas as pl
from jax.experimental.pallas import tpu as pltpu
from jax.experimental.pallas import tpu_sc as plsc

N = 10000
E = 320000
D = 128
H = 8
K = 16
FFN = 512

NC = 2
NS = 16
NW = NC * NS
GRP = 64
BLK = 512
N_PAD = 10240
EPW = 10240
E_PAD = NW * EPW
STRIPE = N_PAD // NS
NSTRIPE = N // NS
DD = D + 16
NEG = -3.0e38

_SC_PARAMS = pltpu.CompilerParams(
    use_tc_tiling_on_sc=False, needs_layout_passes=False)

_GDN = lax.GatherDimensionNumbers(
    offset_dims=(), collapsed_slice_dims=(0,), start_index_map=(0,))


def _dg(x, idx):
    return lax.gather(x, idx[:, None], _GDN, slice_sizes=(1,),
                      mode=lax.GatherScatterMode.PROMISE_IN_BOUNDS)


def _proj_body(x_ref, wf_ref, a_ref, wh_ref, ea_ref):
    x = x_ref[...]
    wh = jnp.dot(x, wf_ref[...], preferred_element_type=jnp.float32)
    wh_ref[...] = wh
    ea_ref[...] = jnp.dot(wh, a_ref[...], preferred_element_type=jnp.float32)


def _proj(x, w_flat, a_mat):
    R = 400
    return pl.pallas_call(
        _proj_body,
        grid=(N // R,),
        in_specs=[
            pl.BlockSpec((R, D), lambda i: (i, 0)),
            pl.BlockSpec((D, D), lambda i: (0, 0)),
            pl.BlockSpec((D, 2 * H), lambda i: (0, 0)),
        ],
        out_specs=[
            pl.BlockSpec((R, D), lambda i: (i, 0)),
            pl.BlockSpec((R, 2 * H), lambda i: (i, 0)),
        ],
        out_shape=[
            jax.ShapeDtypeStruct((N, D), jnp.float32),
            jax.ShapeDtypeStruct((N, 2 * H), jnp.float32),
        ],
    )(x, w_flat, a_mat)


def _segmax_body(src2_hbm, dst2_hbm, ea_hbm, e_out, mparts_out, m0_out,
                 m1_out, macc, sidxb, didxb, rs0, rd0, rs1, rd1, ebuf,
                 mbufa, mbufb, tbuf, sem0, sem1):
    c = lax.axis_index("c")
    s_ = lax.axis_index("s")
    wid = c * NS + s_
    lo = wid * EPW
    nblk = jnp.maximum(0, jnp.minimum(EPW, E - lo)) // BLK

    lane = lax.iota(jnp.int32, 16)
    hsel = lane & 7
    sh8 = (lane + 8) & 15
    lane8 = lane < 8
    neg16 = jnp.full((16,), NEG, jnp.float32)

    def _init(j, _):
        for r in range(H):
            macc[r, pl.ds(j * 16, 16)] = neg16
        return 0
    lax.fori_loop(0, N_PAD // 16, _init, 0)

    def _issue(g, rs, rd, sem):
        pltpu.async_copy(ea_hbm.at[sidxb.at[g]], rs, sem)
        pltpu.async_copy(ea_hbm.at[didxb.at[g]], rd, sem)

    def _wait(g, rs, rd, sem):
        pltpu.make_async_copy(ea_hbm.at[sidxb.at[g]], rs, sem).wait()
        pltpu.make_async_copy(ea_hbm.at[didxb.at[g]], rd, sem).wait()

    def _consume(g, rs, rd):
        def _sub(jj, _):
            dvec = didxb[g, pl.ds(jj * 16, 16)]
            for jl in range(16):
                row = jj * 16 + jl
                ef = rs[row, :] + _dg(rd[row, :], sh8)
                lk = jnp.where(ef >= 0.0, ef, 0.2 * ef)
                ebuf[g * GRP + row, :] = lk
                dsp = _dg(dvec, jnp.full((16,), jl, jnp.int32))
                cur = plsc.load_gather(macc, [hsel, dsp])
                plsc.store_scatter(macc, [hsel, dsp],
                                   jnp.maximum(cur, lk), mask=lane8)
            return 0
        lax.fori_loop(0, GRP // 16, _sub, 0)

    def _blk(b, _):
        row0 = lo // GRP + b * (BLK // GRP)
        pltpu.sync_copy(src2_hbm.at[pl.ds(row0, BLK // GRP)], sidxb)
        pltpu.sync_copy(dst2_hbm.at[pl.ds(row0, BLK // GRP)], didxb)
        _issue(0, rs0, rd0, sem0)
        _issue(1, rs1, rd1, sem1)

        def _pair(p, _):
            g0 = 2 * p
            _wait(g0, rs0, rd0, sem0)
            _consume(g0, rs0, rd0)

            @pl.when(p < (BLK // GRP) // 2 - 1)
            def _():
                _issue(g0 + 2, rs0, rd0, sem0)

            g1 = 2 * p + 1
            _wait(g1, rs1, rd1, sem1)
            _consume(g1, rs1, rd1)

            @pl.when(p < (BLK // GRP) // 2 - 1)
            def _():
                _issue(g1 + 2, rs1, rd1, sem1)
            return 0
        lax.fori_loop(0, (BLK // GRP) // 2, _pair, 0)
        pltpu.sync_copy(ebuf, e_out.at[pl.ds(lo + b * BLK, BLK)])
        return 0
    lax.fori_loop(0, nblk, _blk, 0)

    pltpu.sync_copy(macc, mparts_out.at[wid])
    plsc.subcore_barrier()

    sbase = s_ * STRIPE
    pltpu.sync_copy(mparts_out.at[c * NS, :, pl.ds(sbase, STRIPE)], mbufa)
    for p in range(1, NS):
        pltpu.sync_copy(mparts_out.at[c * NS + p, :, pl.ds(sbase, STRIPE)],
                        mbufb)

        def _mrg(j, _):
            for r in range(H):
                a = mbufa[r, pl.ds(j * 16, 16)]
                b = mbufb[r, pl.ds(j * 16, 16)]
                mbufa[r, pl.ds(j * 16, 16)] = jnp.maximum(a, b)
            return 0
        lax.fori_loop(0, STRIPE // 16, _mrg, 0)

    def _tr(nb, _):
        for jl in range(16):
            node = nb * 16 + jl
            vec = plsc.load_gather(
                mbufa, [hsel, jnp.full((16,), node, jnp.int32)])
            tbuf[jl, :] = vec

        @pl.when(c == 0)
        def _():
            pltpu.sync_copy(tbuf, m0_out.at[pl.ds(sbase + nb * 16, 16)])

        @pl.when(c == 1)
        def _():
            pltpu.sync_copy(tbuf, m1_out.at[pl.ds(sbase + nb * 16, 16)])
        return 0
    lax.fori_loop(0, STRIPE // 16, _tr, 0)


def _sc_segmax(src2, dst2, e_all):
    mesh = plsc.VectorSubcoreMesh(core_axis_name="c", subcore_axis_name="s")
    f = pl.kernel(
        _segmax_body,
        out_type=[
            jax.ShapeDtypeStruct((E_PAD, 16), jnp.float32),
            jax.ShapeDtypeStruct((NW, H, N_PAD), jnp.float32),
            jax.ShapeDtypeStruct((N_PAD, 16), jnp.float32),
            jax.ShapeDtypeStruct((N_PAD, 16), jnp.float32),
        ],
        mesh=mesh,
        compiler_params=_SC_PARAMS,
        scratch_types=[
            pltpu.VMEM((H, N_PAD), jnp.float32),
            pltpu.VMEM((BLK // GRP, GRP), jnp.int32),
            pltpu.VMEM((BLK // GRP, GRP), jnp.int32),
            pltpu.VMEM((GRP, 16), jnp.float32),
            pltpu.VMEM((GRP, 16), jnp.float32),
            pltpu.VMEM((GRP, 16), jnp.float32),
            pltpu.VMEM((GRP, 16), jnp.float32),
            pltpu.VMEM((BLK, 16), jnp.float32),
            pltpu.VMEM((H, STRIPE), jnp.float32),
            pltpu.VMEM((H, STRIPE), jnp.float32),
            pltpu.VMEM((16, 16), jnp.float32),
            pltpu.SemaphoreType.DMA,
            pltpu.SemaphoreType.DMA,
        ],
    )
    e_edge, _mparts, m0, m1 = f(src2, dst2, e_all)
    return e_edge, m0, m1


def _mmax_body(a_ref, b_ref, o_ref):
    o_ref[...] = jnp.maximum(a_ref[...], b_ref[...])


def _mmax(m0, m1):
    R = 512
    return pl.pallas_call(
        _mmax_body,
        grid=(N_PAD // R,),
        in_specs=[
            pl.BlockSpec((R, 16), lambda i: (i, 0)),
            pl.BlockSpec((R, 16), lambda i: (i, 0)),
        ],
        out_specs=pl.BlockSpec((R, 16), lambda i: (i, 0)),
        out_shape=jax.ShapeDtypeStruct((N_PAD, 16), jnp.float32),
    )(m0, m1)


def _msg_body(src2_hbm, dst2_hbm, e_hbm, m_hbm, wh_hbm, num_out,
              sidxb, didxb, wh0, eb0, mb0, wh1, eb1, mb1, nstage,
              num_sh, sem0, sem1):
    c = lax.axis_index("c")
    s_ = lax.axis_index("s")
    wid = c * NS + s_
    lo = wid * EPW
    nblk = jnp.maximum(0, jnp.minimum(EPW, E - lo)) // BLK

    z16 = jnp.zeros((16,), jnp.float32)

    def _z(r, _):
        for cb in range(DD // 16):
            nstage[r, pl.ds(cb * 16, 16)] = z16
        return 0
    lax.fori_loop(0, GRP, _z, 0)
    nbase = s_ * NSTRIPE
    for t in range(NSTRIPE // GRP):
        pltpu.sync_copy(nstage, num_sh.at[pl.ds(nbase + t * GRP, GRP)])
    pltpu.sync_copy(nstage.at[pl.ds(0, NSTRIPE % GRP)],
                    num_sh.at[pl.ds(nbase + (NSTRIPE // GRP) * GRP,
                                    NSTRIPE % GRP)])
    plsc.subcore_barrier()

    def _issue(b, g, whb, eb, mb, sem):
        pltpu.async_copy(wh_hbm.at[sidxb.at[g]], whb, sem)
        pltpu.async_copy(e_hbm.at[pl.ds(lo + b * BLK + g * GRP, GRP)], eb,
                         sem)
        pltpu.async_copy(m_hbm.at[didxb.at[g]], mb, sem)

    def _wait(b, g, whb, eb, mb, sem):
        pltpu.make_async_copy(wh_hbm.at[sidxb.at[g]], whb, sem).wait()
        pltpu.make_async_copy(
            e_hbm.at[pl.ds(lo + b * BLK + g * GRP, GRP)], eb, sem).wait()
        pltpu.make_async_copy(m_hbm.at[didxb.at[g]], mb, sem).wait()

    def _consume(g, whb, eb, mb):
        def _sub(jj, _):
            for jl in range(16):
                row = jj * 16 + jl
                ev = eb[row, :]
                mv = mb[row, :]
                ex = jnp.exp(jnp.minimum(ev - mv, 0.0))
                nstage[row, pl.ds(D, 16)] = ex
                for h in range(H):
                    exh = _dg(ex, jnp.full((16,), h, jnp.int32))
                    nstage[row, pl.ds(h * 16, 16)] = (
                        whb[row, pl.ds(h * 16, 16)] * exh)
            return 0
        lax.fori_loop(0, GRP // 16, _sub, 0)
        pltpu.sync_copy(nstage, num_sh.at[didxb.at[g]], add=True)

    def _blk(b, _):
        row0 = lo // GRP + b * (BLK // GRP)
        pltpu.sync_copy(src2_hbm.at[pl.ds(row0, BLK // GRP)], sidxb)
        pltpu.sync_copy(dst2_hbm.at[pl.ds(row0, BLK // GRP)], didxb)
        _issue(b, 0, wh0, eb0, mb0, sem0)
        _issue(b, 1, wh1, eb1, mb1, sem1)

        def _pair(p, _):
            g0 = 2 * p
            _wait(b, g0, wh0, eb0, mb0, sem0)
            _consume(g0, wh0, eb0, mb0)

            @pl.when(p < (BLK // GRP) // 2 - 1)
            def _():
                _issue(b, g0 + 2, wh0, eb0, mb0, sem0)

            g1 = 2 * p + 1
            _wait(b, g1, wh1, eb1, mb1, sem1)
            _consume(g1, wh1, eb1, mb1)

            @pl.when(p < (BLK // GRP) // 2 - 1)
            def _():
                _issue(b, g1 + 2, wh1, eb1, mb1, sem1)
            return 0
        lax.fori_loop(0, (BLK // GRP) // 2, _pair, 0)
        return 0
    lax.fori_loop(0, nblk, _blk, 0)

    plsc.subcore_barrier()
    @pl.when(c == 0)
    def _():
        pltpu.sync_copy(num_sh.at[pl.ds(nbase, NSTRIPE)],
                        num_out.at[0, pl.ds(nbase, NSTRIPE)])

    @pl.when(c == 1)
    def _():
        pltpu.sync_copy(num_sh.at[pl.ds(nbase, NSTRIPE)],
                        num_out.at[1, pl.ds(nbase, NSTRIPE)])


def _sc_msg(src2, dst2, e_edge, m, wh):
    mesh = plsc.VectorSubcoreMesh(core_axis_name="c", subcore_axis_name="s")
    f = pl.kernel(
        _msg_body,
        out_type=jax.ShapeDtypeStruct((NC, N, DD), jnp.float32),
        mesh=mesh,
        compiler_params=_SC_PARAMS,
        scratch_types=[
            pltpu.VMEM((BLK // GRP, GRP), jnp.int32),
            pltpu.VMEM((BLK // GRP, GRP), jnp.int32),
            pltpu.VMEM((GRP, D), jnp.float32),
            pltpu.VMEM((GRP, 16), jnp.float32),
            pltpu.VMEM((GRP, 16), jnp.float32),
            pltpu.VMEM((GRP, D), jnp.float32),
            pltpu.VMEM((GRP, 16), jnp.float32),
            pltpu.VMEM((GRP, 16), jnp.float32),
            pltpu.VMEM((GRP, DD), jnp.float32),
            pltpu.VMEM_SHARED((N, DD), jnp.float32),
            pltpu.SemaphoreType.DMA,
            pltpu.SemaphoreType.DMA,
        ],
    )
    return f(src2, dst2, e_edge, m, wh)


def _ffn_body(nd_ref, s_ref, lng_ref, lnb_ref,
              w1_ref, b1_ref, w2_ref, b2_ref, out_ref):
    nd = nd_ref[0] + nd_ref[1]
    num = nd[:, 0:D]
    den8 = nd[:, D:D + H]
    rr = lax.broadcasted_iota(jnp.int32, (H, D), 0)
    cc = lax.broadcasted_iota(jnp.int32, (H, D), 1)
    bmat = (rr == cc // K).astype(jnp.float32)
    denb = jnp.dot(den8, bmat, preferred_element_type=jnp.float32)
    gat = num / (denb + 1e-10)
    gat = jnp.where(gat > 0.0, gat, jnp.exp(jnp.minimum(gat, 0.0)) - 1.0)
    h = gat + s_ref[...]
    mu = jnp.mean(h, axis=-1, keepdims=True)
    var = jnp.mean((h - mu) * (h - mu), axis=-1, keepdims=True)
    hn = (h - mu) / jnp.sqrt(var + 1e-6) * lng_ref[...] + lnb_ref[...]
    inter = jnp.maximum(
        jnp.dot(hn, w1_ref[...], preferred_element_type=jnp.float32)
        + b1_ref[...], 0.0)
    out_ref[...] = (jnp.dot(inter, w2_ref[...],
                            preferred_element_type=jnp.float32)
                    + b2_ref[...] + h)


def _ffn_tc(numden, s, ln_g, ln_b, w1, b1, w2, b2):
    R = 400
    return pl.pallas_call(
        _ffn_body,
        grid=(N // R,),
        in_specs=[
            pl.BlockSpec((NC, R, DD), lambda i: (0, i, 0)),
            pl.BlockSpec((R, D), lambda i: (i, 0)),
            pl.BlockSpec((1, D), lambda i: (0, 0)),
            pl.BlockSpec((1, D), lambda i: (0, 0)),
            pl.BlockSpec((D, FFN), lambda i: (0, 0)),
            pl.BlockSpec((1, FFN), lambda i: (0, 0)),
            pl.BlockSpec((FFN, D), lambda i: (0, 0)),
            pl.BlockSpec((1, D), lambda i: (0, 0)),
        ],
        out_specs=pl.BlockSpec((R, D), lambda i: (i, 0)),
        out_shape=jax.ShapeDtypeStruct((N, D), jnp.float32),
    )(numden, s, ln_g.reshape(1, D), ln_b.reshape(1, D),
      w1, b1.reshape(1, FFN), w2, b2.reshape(1, D))


def kernel(w, s, edge_index, W, a_src, a_dst, ln_g, ln_b, W1, b1, W2, b2):
    w_flat = W.transpose(1, 0, 2).reshape(D, D)
    eye8 = jnp.eye(H, dtype=jnp.float32)
    a1 = (a_src[:, :, None] * eye8[:, None, :]).reshape(D, H)
    a2 = (a_dst[:, :, None] * eye8[:, None, :]).reshape(D, H)
    a_mat = jnp.concatenate([a1, a2], axis=1)

    src = edge_index[0].astype(jnp.int32)
    dst = edge_index[1].astype(jnp.int32)
    pad = jnp.zeros((E_PAD - E,), jnp.int32)
    src2 = jnp.concatenate([src, pad]).reshape(E_PAD // GRP, GRP)
    dst2 = jnp.concatenate([dst, pad]).reshape(E_PAD // GRP, GRP)

    wh, e_all = _proj(w, w_flat, a_mat)
    e_edge, m0, m1 = _sc_segmax(src2, dst2, e_all)
    m = _mmax(m0, m1)
    numden = _sc_msg(src2, dst2, e_edge, m, wh)
    return _ffn_tc(numden, s, ln_g, ln_b, W1, b1, W2, b2)


# speedup vs baseline: 39.0784x; 1.0045x over previous
import jax
import jax.numpy as jnp
from jax import lax
from jax.experimental import pallas as pl
from jax.experimental.pallas import tpu as pltpu
from jax.experimental.pallas import tpu_sc as plsc

N = 10000
E = 320000
D = 128
H = 8
K = 16
FFN = 512

NC = 2
NS = 16
NW = NC * NS
GRP = 64
BLK = 512
N_PAD = 10240
EPW = 10240
E_PAD = NW * EPW
STRIPE = N_PAD // NS
NSTRIPE = N // NS
DD = D + 16
NEG = -3.0e38

_SC_PARAMS = pltpu.CompilerParams(
    use_tc_tiling_on_sc=False, needs_layout_passes=False)

_GDN = lax.GatherDimensionNumbers(
    offset_dims=(), collapsed_slice_dims=(0,), start_index_map=(0,))


def _dg(x, idx):
    return lax.gather(x, idx[:, None], _GDN, slice_sizes=(1,),
                      mode=lax.GatherScatterMode.PROMISE_IN_BOUNDS)


def _proj_body(x_ref, wf_ref, a_ref, wh_ref, ea_ref):
    x = x_ref[...]
    wh = jnp.dot(x, wf_ref[...], preferred_element_type=jnp.float32)
    wh_ref[...] = wh
    ea_ref[...] = jnp.dot(wh, a_ref[...], preferred_element_type=jnp.float32)


def _proj(x, w_flat, a_mat):
    R = 400
    return pl.pallas_call(
        _proj_body,
        grid=(N // R,),
        in_specs=[
            pl.BlockSpec((R, D), lambda i: (i, 0)),
            pl.BlockSpec((D, D), lambda i: (0, 0)),
            pl.BlockSpec((D, 2 * H), lambda i: (0, 0)),
        ],
        out_specs=[
            pl.BlockSpec((R, D), lambda i: (i, 0)),
            pl.BlockSpec((R, 2 * H), lambda i: (i, 0)),
        ],
        out_shape=[
            jax.ShapeDtypeStruct((N, D), jnp.float32),
            jax.ShapeDtypeStruct((N, 2 * H), jnp.float32),
        ],
    )(x, w_flat, a_mat)


def _segmax_body(src2_hbm, dst2_hbm, ea_hbm, e_out, mparts_out, m0_out,
                 m1_out, macc, sidxb, didxb, rs0, rd0, rs1, rd1, ebuf,
                 mbufa, mbufb, tbuf, sem0, sem1):
    c = lax.axis_index("c")
    s_ = lax.axis_index("s")
    wid = c * NS + s_
    lo = wid * EPW
    nblk = jnp.maximum(0, jnp.minimum(EPW, E - lo)) // BLK

    lane = lax.iota(jnp.int32, 16)
    hsel = lane & 7
    sh8 = (lane + 8) & 15
    lane8 = lane < 8
    neg16 = jnp.full((16,), NEG, jnp.float32)

    def _init(j, _):
        for r in range(H):
            macc[r, pl.ds(j * 16, 16)] = neg16
        return 0
    lax.fori_loop(0, N_PAD // 16, _init, 0)

    def _issue(g, rs, rd, sem):
        pltpu.async_copy(ea_hbm.at[sidxb.at[g]], rs, sem)
        pltpu.async_copy(ea_hbm.at[didxb.at[g]], rd, sem)

    def _wait(g, rs, rd, sem):
        pltpu.make_async_copy(ea_hbm.at[sidxb.at[g]], rs, sem).wait()
        pltpu.make_async_copy(ea_hbm.at[didxb.at[g]], rd, sem).wait()

    def _consume(g, rs, rd):
        def _sub(jj, _):
            dvec = didxb[g, pl.ds(jj * 16, 16)]
            for jl in range(16):
                row = jj * 16 + jl
                ef = rs[row, :] + _dg(rd[row, :], sh8)
                lk = jnp.where(ef >= 0.0, ef, 0.2 * ef)
                ebuf[g * GRP + row, :] = lk
                dsp = _dg(dvec, jnp.full((16,), jl, jnp.int32))
                cur = plsc.load_gather(macc, [hsel, dsp])
                plsc.store_scatter(macc, [hsel, dsp],
                                   jnp.maximum(cur, lk), mask=lane8)
            return 0
        lax.fori_loop(0, GRP // 16, _sub, 0)

    def _blk(b, _):
        row0 = lo // GRP + b * (BLK // GRP)
        pltpu.sync_copy(src2_hbm.at[pl.ds(row0, BLK // GRP)], sidxb)
        pltpu.sync_copy(dst2_hbm.at[pl.ds(row0, BLK // GRP)], didxb)
        _issue(0, rs0, rd0, sem0)
        _issue(1, rs1, rd1, sem1)

        def _pair(p, _):
            g0 = 2 * p
            _wait(g0, rs0, rd0, sem0)
            _consume(g0, rs0, rd0)

            @pl.when(p < (BLK // GRP) // 2 - 1)
            def _():
                _issue(g0 + 2, rs0, rd0, sem0)

            g1 = 2 * p + 1
            _wait(g1, rs1, rd1, sem1)
            _consume(g1, rs1, rd1)

            @pl.when(p < (BLK // GRP) // 2 - 1)
            def _():
                _issue(g1 + 2, rs1, rd1, sem1)
            return 0
        lax.fori_loop(0, (BLK // GRP) // 2, _pair, 0)
        pltpu.sync_copy(ebuf, e_out.at[pl.ds(lo + b * BLK, BLK)])
        return 0
    lax.fori_loop(0, nblk, _blk, 0)

    pltpu.sync_copy(macc, mparts_out.at[wid])
    plsc.subcore_barrier()

    sbase = s_ * STRIPE
    pltpu.sync_copy(mparts_out.at[c * NS, :, pl.ds(sbase, STRIPE)], mbufa)
    for p in range(1, NS):
        pltpu.sync_copy(mparts_out.at[c * NS + p, :, pl.ds(sbase, STRIPE)],
                        mbufb)

        def _mrg(j, _):
            for r in range(H):
                a = mbufa[r, pl.ds(j * 16, 16)]
                b = mbufb[r, pl.ds(j * 16, 16)]
                mbufa[r, pl.ds(j * 16, 16)] = jnp.maximum(a, b)
            return 0
        lax.fori_loop(0, STRIPE // 16, _mrg, 0)

    def _tr(nb, _):
        for jl in range(16):
            node = nb * 16 + jl
            vec = plsc.load_gather(
                mbufa, [hsel, jnp.full((16,), node, jnp.int32)])
            tbuf[jl, :] = vec

        @pl.when(c == 0)
        def _():
            pltpu.sync_copy(tbuf, m0_out.at[pl.ds(sbase + nb * 16, 16)])

        @pl.when(c == 1)
        def _():
            pltpu.sync_copy(tbuf, m1_out.at[pl.ds(sbase + nb * 16, 16)])
        return 0
    lax.fori_loop(0, STRIPE // 16, _tr, 0)


def _sc_segmax(src2, dst2, e_all):
    mesh = plsc.VectorSubcoreMesh(core_axis_name="c", subcore_axis_name="s")
    f = pl.kernel(
        _segmax_body,
        out_type=[
            jax.ShapeDtypeStruct((E_PAD, 16), jnp.float32),
            jax.ShapeDtypeStruct((NW, H, N_PAD), jnp.float32),
            jax.ShapeDtypeStruct((N_PAD, 16), jnp.float32),
            jax.ShapeDtypeStruct((N_PAD, 16), jnp.float32),
        ],
        mesh=mesh,
        compiler_params=_SC_PARAMS,
        scratch_types=[
            pltpu.VMEM((H, N_PAD), jnp.float32),
            pltpu.VMEM((BLK // GRP, GRP), jnp.int32),
            pltpu.VMEM((BLK // GRP, GRP), jnp.int32),
            pltpu.VMEM((GRP, 16), jnp.float32),
            pltpu.VMEM((GRP, 16), jnp.float32),
            pltpu.VMEM((GRP, 16), jnp.float32),
            pltpu.VMEM((GRP, 16), jnp.float32),
            pltpu.VMEM((BLK, 16), jnp.float32),
            pltpu.VMEM((H, STRIPE), jnp.float32),
            pltpu.VMEM((H, STRIPE), jnp.float32),
            pltpu.VMEM((16, 16), jnp.float32),
            pltpu.SemaphoreType.DMA,
            pltpu.SemaphoreType.DMA,
        ],
    )
    e_edge, _mparts, m0, m1 = f(src2, dst2, e_all)
    return e_edge, m0, m1


def _mmax_body(a_ref, b_ref, o_ref):
    o_ref[...] = jnp.maximum(a_ref[...], b_ref[...])


def _mmax(m0, m1):
    R = 512
    return pl.pallas_call(
        _mmax_body,
        grid=(N_PAD // R,),
        in_specs=[
            pl.BlockSpec((R, 16), lambda i: (i, 0)),
            pl.BlockSpec((R, 16), lambda i: (i, 0)),
        ],
        out_specs=pl.BlockSpec((R, 16), lambda i: (i, 0)),
        out_shape=jax.ShapeDtypeStruct((N_PAD, 16), jnp.float32),
    )(m0, m1)


def _msg_body(src2_hbm, dst2p_hbm, e_hbm, m_hbm, wh_hbm, num_out,
              sidxb, didxp, wh0, eb0, mb0, wh1, eb1, mb1, nstage,
              num_sh, sem0, sem1):
    c = lax.axis_index("c")
    s_ = lax.axis_index("s")
    wid = c * NS + s_
    lo = wid * EPW
    nblk = jnp.maximum(0, jnp.minimum(EPW, E - lo)) // BLK

    z16 = jnp.zeros((16,), jnp.float32)

    def _z(r, _):
        for cb in range(DD // 16):
            nstage[r, pl.ds(cb * 16, 16)] = z16
        return 0
    lax.fori_loop(0, 2 * GRP, _z, 0)
    nbase = s_ * NSTRIPE
    for t in range(NSTRIPE // (2 * GRP)):
        pltpu.sync_copy(nstage,
                        num_sh.at[pl.ds(nbase + t * 2 * GRP, 2 * GRP)])
    pltpu.sync_copy(nstage.at[pl.ds(0, NSTRIPE % (2 * GRP))],
                    num_sh.at[pl.ds(nbase + (NSTRIPE // (2 * GRP)) * 2 * GRP,
                                    NSTRIPE % (2 * GRP))])
    plsc.subcore_barrier()

    def _issue(b, prow, half, whb, eb, mb, sem):
        g = 2 * prow + half
        pltpu.async_copy(wh_hbm.at[sidxb.at[g]], whb, sem)
        pltpu.async_copy(e_hbm.at[pl.ds(lo + b * BLK + g * GRP, GRP)], eb,
                         sem)
        pltpu.async_copy(m_hbm.at[didxp.at[prow, pl.ds(half * GRP, GRP)]],
                         mb, sem)

    def _wait(b, prow, half, whb, eb, mb, sem):
        g = 2 * prow + half
        pltpu.make_async_copy(wh_hbm.at[sidxb.at[g]], whb, sem).wait()
        pltpu.make_async_copy(
            e_hbm.at[pl.ds(lo + b * BLK + g * GRP, GRP)], eb, sem).wait()
        pltpu.make_async_copy(
            m_hbm.at[didxp.at[prow, pl.ds(half * GRP, GRP)]], mb,
            sem).wait()

    def _consume(whb, eb, mb, half):
        def _sub(jj, _):
            for jl in range(16):
                row = jj * 16 + jl
                ev = eb[row, :]
                mv = mb[row, :]
                ex = jnp.exp(jnp.minimum(ev - mv, 0.0))
                nstage[half * GRP + row, pl.ds(D, 16)] = ex
                for h in range(H):
                    exh = _dg(ex, jnp.full((16,), h, jnp.int32))
                    nstage[half * GRP + row, pl.ds(h * 16, 16)] = (
                        whb[row, pl.ds(h * 16, 16)] * exh)
            return 0
        lax.fori_loop(0, GRP // 16, _sub, 0)

    def _blk(b, _):
        row0 = lo // GRP + b * (BLK // GRP)
        pltpu.sync_copy(src2_hbm.at[pl.ds(row0, BLK // GRP)], sidxb)
        row0p = lo // (2 * GRP) + b * (BLK // (2 * GRP))
        pltpu.sync_copy(dst2p_hbm.at[pl.ds(row0p, BLK // (2 * GRP))], didxp)
        _issue(b, 0, 0, wh0, eb0, mb0, sem0)
        _issue(b, 0, 1, wh1, eb1, mb1, sem1)

        def _pair(p, _):
            _wait(b, p, 0, wh0, eb0, mb0, sem0)
            _consume(wh0, eb0, mb0, 0)

            @pl.when(p < (BLK // GRP) // 2 - 1)
            def _():
                _issue(b, p + 1, 0, wh0, eb0, mb0, sem0)

            _wait(b, p, 1, wh1, eb1, mb1, sem1)
            _consume(wh1, eb1, mb1, 1)

            @pl.when(p < (BLK // GRP) // 2 - 1)
            def _():
                _issue(b, p + 1, 1, wh1, eb1, mb1, sem1)

            pltpu.sync_copy(nstage, num_sh.at[didxp.at[p]], add=True)
            return 0
        lax.fori_loop(0, (BLK // GRP) // 2, _pair, 0)
        return 0
    lax.fori_loop(0, nblk, _blk, 0)

    plsc.subcore_barrier()
    @pl.when(c == 0)
    def _():
        pltpu.sync_copy(num_sh.at[pl.ds(nbase, NSTRIPE)],
                        num_out.at[0, pl.ds(nbase, NSTRIPE)])

    @pl.when(c == 1)
    def _():
        pltpu.sync_copy(num_sh.at[pl.ds(nbase, NSTRIPE)],
                        num_out.at[1, pl.ds(nbase, NSTRIPE)])


def _sc_msg(src2, dst2p, e_edge, m, wh):
    mesh = plsc.VectorSubcoreMesh(core_axis_name="c", subcore_axis_name="s")
    f = pl.kernel(
        _msg_body,
        out_type=jax.ShapeDtypeStruct((NC, N, DD), jnp.float32),
        mesh=mesh,
        compiler_params=_SC_PARAMS,
        scratch_types=[
            pltpu.VMEM((BLK // GRP, GRP), jnp.int32),
            pltpu.VMEM((BLK // (2 * GRP), 2 * GRP), jnp.int32),
            pltpu.VMEM((GRP, D), jnp.float32),
            pltpu.VMEM((GRP, 16), jnp.float32),
            pltpu.VMEM((GRP, 16), jnp.float32),
            pltpu.VMEM((GRP, D), jnp.float32),
            pltpu.VMEM((GRP, 16), jnp.float32),
            pltpu.VMEM((GRP, 16), jnp.float32),
            pltpu.VMEM((2 * GRP, DD), jnp.float32),
            pltpu.VMEM_SHARED((N, DD), jnp.float32),
            pltpu.SemaphoreType.DMA,
            pltpu.SemaphoreType.DMA,
        ],
    )
    return f(src2, dst2p, e_edge, m, wh)


def _ffn_body(nd_ref, s_ref, lng_ref, lnb_ref,
              w1_ref, b1_ref, w2_ref, b2_ref, out_ref):
    nd = nd_ref[0] + nd_ref[1]
    num = nd[:, 0:D]
    den8 = nd[:, D:D + H]
    rr = lax.broadcasted_iota(jnp.int32, (H, D), 0)
    cc = lax.broadcasted_iota(jnp.int32, (H, D), 1)
    bmat = (rr == cc // K).astype(jnp.float32)
    denb = jnp.dot(den8, bmat, preferred_element_type=jnp.float32)
    gat = num / (denb + 1e-10)
    gat = jnp.where(gat > 0.0, gat, jnp.exp(jnp.minimum(gat, 0.0)) - 1.0)
    h = gat + s_ref[...]
    mu = jnp.mean(h, axis=-1, keepdims=True)
    var = jnp.mean((h - mu) * (h - mu), axis=-1, keepdims=True)
    hn = (h - mu) / jnp.sqrt(var + 1e-6) * lng_ref[...] + lnb_ref[...]
    inter = jnp.maximum(
        jnp.dot(hn, w1_ref[...], preferred_element_type=jnp.float32)
        + b1_ref[...], 0.0)
    out_ref[...] = (jnp.dot(inter, w2_ref[...],
                            preferred_element_type=jnp.float32)
                    + b2_ref[...] + h)


def _ffn_tc(numden, s, ln_g, ln_b, w1, b1, w2, b2):
    R = 400
    return pl.pallas_call(
        _ffn_body,
        grid=(N // R,),
        in_specs=[
            pl.BlockSpec((NC, R, DD), lambda i: (0, i, 0)),
            pl.BlockSpec((R, D), lambda i: (i, 0)),
            pl.BlockSpec((1, D), lambda i: (0, 0)),
            pl.BlockSpec((1, D), lambda i: (0, 0)),
            pl.BlockSpec((D, FFN), lambda i: (0, 0)),
            pl.BlockSpec((1, FFN), lambda i: (0, 0)),
            pl.BlockSpec((FFN, D), lambda i: (0, 0)),
            pl.BlockSpec((1, D), lambda i: (0, 0)),
        ],
        out_specs=pl.BlockSpec((R, D), lambda i: (i, 0)),
        out_shape=jax.ShapeDtypeStruct((N, D), jnp.float32),
    )(numden, s, ln_g.reshape(1, D), ln_b.reshape(1, D),
      w1, b1.reshape(1, FFN), w2, b2.reshape(1, D))


def kernel(w, s, edge_index, W, a_src, a_dst, ln_g, ln_b, W1, b1, W2, b2):
    w_flat = W.transpose(1, 0, 2).reshape(D, D)
    eye8 = jnp.eye(H, dtype=jnp.float32)
    a1 = (a_src[:, :, None] * eye8[:, None, :]).reshape(D, H)
    a2 = (a_dst[:, :, None] * eye8[:, None, :]).reshape(D, H)
    a_mat = jnp.concatenate([a1, a2], axis=1)

    src = edge_index[0].astype(jnp.int32)
    dst = edge_index[1].astype(jnp.int32)
    pad = jnp.zeros((E_PAD - E,), jnp.int32)
    src2 = jnp.concatenate([src, pad]).reshape(E_PAD // GRP, GRP)
    dstp = jnp.concatenate([dst, pad])
    dst2 = dstp.reshape(E_PAD // GRP, GRP)
    dst2p = dstp.reshape(E_PAD // (2 * GRP), 2 * GRP)

    wh, e_all = _proj(w, w_flat, a_mat)
    e_edge, m0, m1 = _sc_segmax(src2, dst2, e_all)
    m = _mmax(m0, m1)
    numden = _sc_msg(src2, dst2p, e_edge, m, wh)
    return _ffn_tc(numden, s, ln_g, ln_b, W1, b1, W2, b2)
